# baseline (speedup 1.0000x reference)
import numpy as np
import jax
import jax.numpy as jnp
from jax import lax
from jax.experimental import pallas as pl
from jax.experimental.pallas import tpu as pltpu
from jax.experimental.pallas import tpu_sc as plsc

_T = 1000


def _make_tables():
    beta = np.linspace(0.0001, 0.02, _T, dtype=np.float32)
    alpha = (1.0 - beta).astype(np.float32)
    alpha_hat = np.cumprod(alpha, dtype=np.float32)
    sa = np.sqrt(alpha_hat).astype(np.float32)
    sb = np.sqrt((1.0 - alpha_hat).astype(np.float32)).astype(np.float32)
    return sa, sb


_SA, _SB = _make_tables()

_B = 256
_N = 3 * 128 * 128
_NW = 32
_RPW = _B // _NW
_CH = _N // 2
_NCH = _RPW * 2
_BM = 8


def _tc_body(ts_ref, sa_ref, sb_ref, x_ref, e_ref, o_ref):
    base = pl.program_id(0) * _BM
    ca, cb = [], []
    for r in range(_BM):
        t = ts_ref[base + r]
        ca.append(sa_ref[t])
        cb.append(sb_ref[t])
    a = jnp.stack(ca).reshape(_BM, 1, 1, 1)
    b = jnp.stack(cb).reshape(_BM, 1, 1, 1)
    o_ref[...] = a * x_ref[...] + b * e_ref[...]


def _sc_body(e_hbm, out2_hbm, v0, v1, v2, sem):
    v = [v0, v1, v2]
    wid = lax.axis_index("s") * 2 + lax.axis_index("c")
    base = wid * _RPW * _N

    def issue_in(k):
        off = base + k * _CH
        pltpu.async_copy(e_hbm.at[pl.ds(off, _CH)], v[k % 3], sem.at[k % 3, 0])

    def wait_in(j):
        off = base + j * _CH
        pltpu.make_async_copy(e_hbm.at[pl.ds(off, _CH)], v[j % 3], sem.at[j % 3, 0]).wait()

    def issue_out(j):
        off = base + j * _CH
        pltpu.async_copy(v[j % 3], out2_hbm.at[pl.ds(off, _CH)], sem.at[j % 3, 1])

    def wait_out(j):
        off = base + j * _CH
        pltpu.make_async_copy(v[j % 3], out2_hbm.at[pl.ds(off, _CH)], sem.at[j % 3, 1]).wait()

    issue_in(0)
    issue_in(1)
    for j in range(_NCH):
        k = j + 2
        if k < _NCH:
            if k >= 3:
                wait_out(k - 3)
            issue_in(k)
        wait_in(j)
        issue_out(j)
    wait_out(_NCH - 3)
    wait_out(_NCH - 2)
    wait_out(_NCH - 1)


def kernel(x0, timesteps, eps):
    B, C, H, W = x0.shape
    ts = timesteps.astype(jnp.int32)
    sa = jnp.asarray(_SA)
    sb = jnp.asarray(_SB)

    mesh = plsc.VectorSubcoreMesh(core_axis_name="c", subcore_axis_name="s")
    sc_copy = pl.kernel(
        _sc_body,
        mesh=mesh,
        out_type=jax.ShapeDtypeStruct((_B * _N,), jnp.float32),
        scratch_types=[
            pltpu.VMEM((_CH,), jnp.float32),
            pltpu.VMEM((_CH,), jnp.float32),
            pltpu.VMEM((_CH,), jnp.float32),
            pltpu.SemaphoreType.DMA((3, 2)),
        ],
    )
    out2 = sc_copy(eps.reshape(_B * _N))

    grid = (B // _BM,)
    smem = pl.BlockSpec(memory_space=pltpu.SMEM)
    blk = pl.BlockSpec((_BM, C, H, W), lambda i: (i, 0, 0, 0))
    out = pl.pallas_call(
        _tc_body,
        grid=grid,
        in_specs=[smem, smem, smem, blk, blk],
        out_specs=blk,
        out_shape=jax.ShapeDtypeStruct((B, C, H, W), jnp.float32),
    )(ts, sa, sb, x0, eps)
    return (out, out2.reshape(x0.shape))
